# Initial kernel scaffold; baseline (speedup 1.0000x reference)
import jax
import jax.numpy as jnp
from jax.experimental import pallas as pl


def kernel(x, edge_index, batch, W1, b1, W2, b2, W3, b3, lin_W, lin_b):
    raise NotImplementedError("write your pallas kernel here")


# speedup vs baseline: 8.7162x; 8.7162x over previous
import functools

import jax
import jax.numpy as jnp
from jax import lax
from jax.experimental import pallas as pl
from jax.experimental.pallas import tpu as pltpu
from jax.experimental.pallas import tpu_sc as plsc

N_NODES = 10000
FEAT = 128
N_GRAPHS = 128
EDGES = 320000

NPAD = 10240
NBLK = NPAD // 128
NTILES = 32
CHUNK = 128
EPT = 10112
EPAD = NTILES * EPT
NCHUNK = EPT // CHUNK
RPT = NPAD // 16

_f32 = jnp.float32


@functools.lru_cache(maxsize=None)
def _sc_kernels():
    mesh = plsc.VectorSubcoreMesh(core_axis_name="c", subcore_axis_name="s")

    @functools.partial(
        pl.kernel,
        out_type=jax.ShapeDtypeStruct((2, NPAD), _f32),
        mesh=mesh,
        scratch_types=[
            pltpu.VMEM_SHARED((NPAD,), _f32),
            pltpu.VMEM((CHUNK,), jnp.int32),
            pltpu.VMEM((CHUNK,), _f32),
            pltpu.VMEM((RPT,), _f32),
        ],
    )
    def deg_kernel(dst_hbm, out_hbm, acc, didx, ones_v, buf):
        cid = lax.axis_index("c")
        sid = lax.axis_index("s")
        wid = cid * 16 + sid
        for k in range(CHUNK // 16):
            ones_v[pl.ds(k * 16, 16)] = jnp.ones((16,), _f32)
        for k in range(RPT // 16):
            buf[pl.ds(k * 16, 16)] = jnp.zeros((16,), _f32)
        pltpu.sync_copy(buf, acc.at[pl.ds(sid * RPT, RPT)])
        plsc.subcore_barrier()
        base = wid * EPT

        @pl.loop(0, NCHUNK)
        def _(c):
            pltpu.sync_copy(dst_hbm.at[pl.ds(base + c * CHUNK, CHUNK)], didx)
            pltpu.sync_copy(ones_v, acc.at[didx], add=True)

        plsc.subcore_barrier()
        pltpu.sync_copy(acc.at[pl.ds(sid * RPT, RPT)], buf)
        pltpu.sync_copy(buf, out_hbm.at[cid, pl.ds(sid * RPT, RPT)])

    @functools.partial(
        pl.kernel,
        out_type=jax.ShapeDtypeStruct((2, NPAD, FEAT), _f32),
        mesh=mesh,
        scratch_types=[
            pltpu.VMEM_SHARED((NPAD, FEAT), _f32),
            pltpu.VMEM((CHUNK,), jnp.int32),
            pltpu.VMEM((CHUNK,), jnp.int32),
            pltpu.VMEM((CHUNK, FEAT), _f32),
            pltpu.VMEM((16, FEAT), _f32),
        ],
    )
    def spmm_kernel(h_hbm, src_hbm, dst_hbm, out_hbm, acc, sidx, didx, rows, zbuf):
        cid = lax.axis_index("c")
        sid = lax.axis_index("s")
        wid = cid * 16 + sid
        for j in range(16):
            for k in range(FEAT // 16):
                zbuf[j, pl.ds(k * 16, 16)] = jnp.zeros((16,), _f32)

        @pl.loop(0, RPT, step=16)
        def _(r):
            pltpu.sync_copy(zbuf, acc.at[pl.ds(sid * RPT + r, 16)])

        plsc.subcore_barrier()
        base = wid * EPT

        @pl.loop(0, NCHUNK)
        def _(c):
            off = base + c * CHUNK
            pltpu.sync_copy(src_hbm.at[pl.ds(off, CHUNK)], sidx)
            pltpu.sync_copy(dst_hbm.at[pl.ds(off, CHUNK)], didx)
            pltpu.sync_copy(h_hbm.at[sidx], rows)
            pltpu.sync_copy(rows, acc.at[didx], add=True)

        plsc.subcore_barrier()

        @pl.loop(0, RPT, step=CHUNK)
        def _(r):
            pltpu.sync_copy(acc.at[pl.ds(sid * RPT + r, CHUNK)], rows)
            pltpu.sync_copy(rows, out_hbm.at[cid, pl.ds(sid * RPT + r, CHUNK)])

    return deg_kernel, spmm_kernel


def _prep_body(degp_ref, x_ref, w_ref, h_ref, dis_ref):
    deg = degp_ref[0] + degp_ref[1] + 1.0
    dis = lax.rsqrt(deg)
    h = jnp.dot(x_ref[...], w_ref[...], preferred_element_type=_f32)
    h_ref[...] = h * dis
    dis_ref[...] = dis


def _combine_body(p_ref, h_ref, dis_ref, b_ref, w_ref, out_ref):
    a = dis_ref[...] * (p_ref[0] + p_ref[1] + h_ref[...]) + b_ref[...]
    a = jnp.maximum(a, 0.0)
    out_ref[...] = jnp.dot(a, w_ref[...], preferred_element_type=_f32) * dis_ref[...]


def _final_body(p_ref, h_ref, dis_ref, b_ref, batch_ref, linw_ref, linb_ref,
                out_ref, sums, cnt):
    i = pl.program_id(0)

    @pl.when(i == 0)
    def _():
        sums[...] = jnp.zeros_like(sums)
        cnt[...] = jnp.zeros_like(cnt)

    a = dis_ref[...] * (p_ref[0] + p_ref[1] + h_ref[...]) + b_ref[...]
    a = jnp.maximum(a, 0.0)
    onehot = (batch_ref[...] == lax.broadcasted_iota(jnp.int32, (1, N_GRAPHS), 1))
    onehot = onehot.astype(_f32)
    dn = (((0,), (0,)), ((), ()))
    sums[...] += lax.dot_general(onehot, a, dn, preferred_element_type=_f32)
    cnt[...] += lax.dot_general(onehot, jnp.ones((128, 1), _f32), dn,
                                preferred_element_type=_f32)

    @pl.when(i == pl.num_programs(0) - 1)
    def _():
        pooled = sums[...] / jnp.maximum(cnt[...], 1.0)
        out_ref[...] = jnp.dot(pooled, linw_ref[...],
                               preferred_element_type=_f32) + linb_ref[...]


def _prep_call(degp3, x_p, W1):
    return pl.pallas_call(
        _prep_body,
        grid=(NBLK,),
        in_specs=[
            pl.BlockSpec((2, 128, 1), lambda i: (0, i, 0)),
            pl.BlockSpec((128, FEAT), lambda i: (i, 0)),
            pl.BlockSpec((FEAT, FEAT), lambda i: (0, 0)),
        ],
        out_specs=[
            pl.BlockSpec((128, FEAT), lambda i: (i, 0)),
            pl.BlockSpec((128, 1), lambda i: (i, 0)),
        ],
        out_shape=[
            jax.ShapeDtypeStruct((NPAD, FEAT), _f32),
            jax.ShapeDtypeStruct((NPAD, 1), _f32),
        ],
    )(degp3, x_p, W1)


def _combine_call(p, h, dis, b_row, W_next):
    return pl.pallas_call(
        _combine_body,
        grid=(NBLK,),
        in_specs=[
            pl.BlockSpec((2, 128, FEAT), lambda i: (0, i, 0)),
            pl.BlockSpec((128, FEAT), lambda i: (i, 0)),
            pl.BlockSpec((128, 1), lambda i: (i, 0)),
            pl.BlockSpec((1, FEAT), lambda i: (0, 0)),
            pl.BlockSpec((FEAT, FEAT), lambda i: (0, 0)),
        ],
        out_specs=pl.BlockSpec((128, FEAT), lambda i: (i, 0)),
        out_shape=jax.ShapeDtypeStruct((NPAD, FEAT), _f32),
    )(p, h, dis, b_row, W_next)


def _final_call(p, h, dis, b_row, batch_p, linw_p, linb_p):
    return pl.pallas_call(
        _final_body,
        grid=(NBLK,),
        in_specs=[
            pl.BlockSpec((2, 128, FEAT), lambda i: (0, i, 0)),
            pl.BlockSpec((128, FEAT), lambda i: (i, 0)),
            pl.BlockSpec((128, 1), lambda i: (i, 0)),
            pl.BlockSpec((1, FEAT), lambda i: (0, 0)),
            pl.BlockSpec((128, 1), lambda i: (i, 0)),
            pl.BlockSpec((FEAT, FEAT), lambda i: (0, 0)),
            pl.BlockSpec((1, FEAT), lambda i: (0, 0)),
        ],
        out_specs=pl.BlockSpec((N_GRAPHS, FEAT), lambda i: (0, 0)),
        out_shape=jax.ShapeDtypeStruct((N_GRAPHS, FEAT), _f32),
        scratch_shapes=[
            pltpu.VMEM((N_GRAPHS, FEAT), _f32),
            pltpu.VMEM((N_GRAPHS, 1), _f32),
        ],
    )(p, h, dis, b_row, batch_p, linw_p, linb_p)


def kernel(x, edge_index, batch, W1, b1, W2, b2, W3, b3, lin_W, lin_b):
    deg_call, spmm_call = _sc_kernels()

    sink = jnp.full((EPAD - EDGES,), N_NODES, jnp.int32)
    src_p = jnp.concatenate([edge_index[0], sink])
    dst_p = jnp.concatenate([edge_index[1], sink])
    x_p = jnp.pad(x, ((0, NPAD - N_NODES), (0, 0)))
    batch_p = jnp.concatenate(
        [batch, jnp.full((NPAD - N_NODES,), N_GRAPHS, batch.dtype)]
    ).reshape(NPAD, 1)
    linw_p = jnp.pad(lin_W, ((0, 0), (0, FEAT - lin_W.shape[1])))
    linb_p = jnp.broadcast_to(lin_b.reshape(1, 1), (1, FEAT))

    degp3 = deg_call(dst_p).reshape(2, NPAD, 1)
    h1, dis = _prep_call(degp3, x_p, W1)
    p1 = spmm_call(h1, src_p, dst_p)
    h2 = _combine_call(p1, h1, dis, b1.reshape(1, FEAT), W2)
    p2 = spmm_call(h2, src_p, dst_p)
    h3 = _combine_call(p2, h2, dis, b2.reshape(1, FEAT), W3)
    p3 = spmm_call(h3, src_p, dst_p)
    outm = _final_call(p3, h3, dis, b3.reshape(1, FEAT), batch_p, linw_p, linb_p)
    return outm[:, :1]
